# Initial kernel scaffold; baseline (speedup 1.0000x reference)
import jax
import jax.numpy as jnp
from jax.experimental import pallas as pl


def kernel(table, index):
    raise NotImplementedError("write your pallas kernel here")


# speedup vs baseline: 190.9374x; 190.9374x over previous
import functools

import jax
import jax.numpy as jnp
from jax import lax
from jax.experimental import pallas as pl
from jax.experimental.pallas import tpu as pltpu
from jax.experimental.pallas import tpu_sc as plsc

ROWS = 16384
COLS = 200
TOTAL = ROWS * COLS
NC = 2
NS = 16
NW = NC * NS
PER_TILE = TOTAL // NW
LANES = 16
CHUNK = 12800
N_CHUNKS = PER_TILE // CHUNK


def _lut_body(table_hbm, idx_hbm, out_hbm, tab_v, idx_v, out_v):
    wid = lax.axis_index("s") * NC + lax.axis_index("c")
    base = wid * PER_TILE
    pltpu.sync_copy(table_hbm, tab_v)

    def chunk_body(c, carry):
        off = base + c * CHUNK
        pltpu.sync_copy(idx_hbm.at[pl.ds(off, CHUNK)], idx_v)

        def vec_body(i, carry2):
            ii = i * LANES
            iv = idx_v[pl.ds(ii, LANES)] + 128
            out_v[pl.ds(ii, LANES)] = plsc.load_gather(tab_v, [iv])
            return carry2

        lax.fori_loop(0, CHUNK // LANES, vec_body, 0, unroll=8)
        pltpu.sync_copy(out_v, out_hbm.at[pl.ds(off, CHUNK)])
        return carry

    lax.fori_loop(0, N_CHUNKS, chunk_body, 0)


@functools.partial(jax.jit, static_argnames=())
def _lut(table, idx_flat):
    mesh = plsc.VectorSubcoreMesh(core_axis_name="c", subcore_axis_name="s")
    f = functools.partial(
        pl.kernel,
        out_type=jax.ShapeDtypeStruct((TOTAL,), jnp.float32),
        mesh=mesh,
        compiler_params=pltpu.CompilerParams(needs_layout_passes=False),
        scratch_types=[
            pltpu.VMEM((256,), jnp.float32),
            pltpu.VMEM((CHUNK,), jnp.int32),
            pltpu.VMEM((CHUNK,), jnp.float32),
        ],
    )(_lut_body)
    return f(table, idx_flat)


def kernel(table, index):
    out_flat = _lut(table, index.reshape(TOTAL))
    out = out_flat.reshape(ROWS, COLS)
    scale = jnp.array([2.0 / 256.0], dtype=jnp.float32)
    return (out, scale)


# speedup vs baseline: 276.6289x; 1.4488x over previous
import functools

import jax
import jax.numpy as jnp
from jax import lax
from jax.experimental import pallas as pl
from jax.experimental.pallas import tpu as pltpu
from jax.experimental.pallas import tpu_sc as plsc

ROWS = 16384
COLS = 200
TOTAL = ROWS * COLS
NC = 2
NS = 16
NW = NC * NS
PER_TILE = TOTAL // NW
LANES = 16
CHUNK = 12800
N_CHUNKS = PER_TILE // CHUNK


def _lut_body(table_hbm, idx_hbm, out_hbm, tab_v, idx_v, out_v):
    wid = lax.axis_index("s") * NC + lax.axis_index("c")
    base = wid * PER_TILE
    pltpu.sync_copy(table_hbm, tab_v)

    def chunk_body(c, carry):
        off = base + c * CHUNK
        pltpu.sync_copy(idx_hbm.at[pl.ds(off, CHUNK)], idx_v)

        @plsc.parallel_loop(0, CHUNK, step=LANES, unroll=8)
        def vec_body(ii):
            iv = idx_v[pl.ds(ii, LANES)] + 128
            out_v[pl.ds(ii, LANES)] = plsc.load_gather(tab_v, [iv])

        pltpu.sync_copy(out_v, out_hbm.at[pl.ds(off, CHUNK)])
        return carry

    lax.fori_loop(0, N_CHUNKS, chunk_body, 0)


@functools.partial(jax.jit, static_argnames=())
def _lut(table, idx_flat):
    mesh = plsc.VectorSubcoreMesh(core_axis_name="c", subcore_axis_name="s")
    f = functools.partial(
        pl.kernel,
        out_type=jax.ShapeDtypeStruct((TOTAL,), jnp.float32),
        mesh=mesh,
        compiler_params=pltpu.CompilerParams(needs_layout_passes=False),
        scratch_types=[
            pltpu.VMEM((256,), jnp.float32),
            pltpu.VMEM((CHUNK,), jnp.int32),
            pltpu.VMEM((CHUNK,), jnp.float32),
        ],
    )(_lut_body)
    return f(table, idx_flat)


def kernel(table, index):
    out_flat = _lut(table, index.reshape(TOTAL))
    out = out_flat.reshape(ROWS, COLS)
    scale = jnp.array([2.0 / 256.0], dtype=jnp.float32)
    return (out, scale)


# speedup vs baseline: 454.0256x; 1.6413x over previous
import functools

import jax
import jax.numpy as jnp
from jax import lax
from jax.experimental import pallas as pl
from jax.experimental.pallas import tpu as pltpu
from jax.experimental.pallas import tpu_sc as plsc

ROWS = 16384
COLS = 200
NC = 2
NS = 16
NW = NC * NS
ROWS_PER_TILE = ROWS // NW
LANES = 16
RBLK = 64
N_BLKS = ROWS_PER_TILE // RBLK
COL_STARTS = tuple(range(0, COLS - LANES, LANES)) + (COLS - LANES,)


def _lut_body(table_hbm, idx_hbm, out_hbm, tab_v, idx_v, out_v):
    wid = lax.axis_index("s") * NC + lax.axis_index("c")
    base = wid * ROWS_PER_TILE
    pltpu.sync_copy(table_hbm, tab_v)

    def blk_body(b, carry):
        row0 = base + b * RBLK
        pltpu.sync_copy(idx_hbm.at[pl.ds(row0, RBLK), :], idx_v)

        @plsc.parallel_loop(0, RBLK, step=1, unroll=2)
        def row_body(r):
            for c in COL_STARTS:
                iv = (idx_v[r, pl.ds(c, LANES)] + 128) & 255
                out_v[r, pl.ds(c, LANES)] = plsc.load_gather(tab_v, [iv])

        pltpu.sync_copy(out_v, out_hbm.at[pl.ds(row0, RBLK), :])
        return carry

    lax.fori_loop(0, N_BLKS, blk_body, 0)


@jax.jit
def _lut(table, index):
    mesh = plsc.VectorSubcoreMesh(core_axis_name="c", subcore_axis_name="s")
    f = functools.partial(
        pl.kernel,
        out_type=jax.ShapeDtypeStruct((ROWS, COLS), jnp.float32),
        mesh=mesh,
        compiler_params=pltpu.CompilerParams(needs_layout_passes=False),
        scratch_types=[
            pltpu.VMEM((256,), jnp.float32),
            pltpu.VMEM((RBLK, COLS), jnp.int32),
            pltpu.VMEM((RBLK, COLS), jnp.float32),
        ],
    )(_lut_body)
    return f(table, index)


def kernel(table, index):
    out = _lut(table, index)
    scale = jnp.array([2.0 / 256.0], dtype=jnp.float32)
    return (out, scale)


# speedup vs baseline: 513.6739x; 1.1314x over previous
import functools

import jax
import jax.numpy as jnp
from jax import lax
from jax.experimental import pallas as pl
from jax.experimental.pallas import tpu as pltpu
from jax.experimental.pallas import tpu_sc as plsc

ROWS = 16384
COLS = 200
NC = 2
NS = 16
NW = NC * NS
ROWS_PER_TILE = ROWS // NW
LANES = 16
RBLK = 64
N_BLKS = ROWS_PER_TILE // RBLK
COL_STARTS = tuple(range(0, COLS - LANES, LANES)) + (COLS - LANES,)


def _lut_body(table_hbm, idx_hbm, out_hbm, tab_v, idx_v, out_v,
              in_sem0, in_sem1, out_sem0, out_sem1):
    in_sems = (in_sem0, in_sem1)
    out_sems = (out_sem0, out_sem1)
    wid = lax.axis_index("s") * NC + lax.axis_index("c")
    base = wid * ROWS_PER_TILE
    pltpu.sync_copy(table_hbm, tab_v)

    def in_copy(b):
        row0 = base + b * RBLK
        return pltpu.make_async_copy(
            idx_hbm.at[pl.ds(row0, RBLK), :], idx_v.at[b % 2], in_sems[b % 2])

    def out_copy(b):
        row0 = base + b * RBLK
        return pltpu.make_async_copy(
            out_v.at[b % 2], out_hbm.at[pl.ds(row0, RBLK), :], out_sems[b % 2])

    in_copy(0).start()
    for b in range(N_BLKS):
        if b + 1 < N_BLKS:
            in_copy(b + 1).start()
        in_copy(b).wait()
        if b >= 2:
            out_copy(b - 2).wait()

        @plsc.parallel_loop(0, RBLK, step=1, unroll=2)
        def row_body(r, _b=b % 2):
            for c in COL_STARTS:
                iv = (idx_v[_b, r, pl.ds(c, LANES)] + 128) & 255
                out_v[_b, r, pl.ds(c, LANES)] = plsc.load_gather(tab_v, [iv])

        out_copy(b).start()
    out_copy(N_BLKS - 2).wait()
    out_copy(N_BLKS - 1).wait()


@jax.jit
def _lut(table, index):
    mesh = plsc.VectorSubcoreMesh(core_axis_name="c", subcore_axis_name="s")
    f = functools.partial(
        pl.kernel,
        out_type=jax.ShapeDtypeStruct((ROWS, COLS), jnp.float32),
        mesh=mesh,
        compiler_params=pltpu.CompilerParams(needs_layout_passes=False),
        scratch_types=[
            pltpu.VMEM((256,), jnp.float32),
            pltpu.VMEM((2, RBLK, COLS), jnp.int32),
            pltpu.VMEM((2, RBLK, COLS), jnp.float32),
            pltpu.SemaphoreType.DMA,
            pltpu.SemaphoreType.DMA,
            pltpu.SemaphoreType.DMA,
            pltpu.SemaphoreType.DMA,
        ],
    )(_lut_body)
    return f(table, index)


def kernel(table, index):
    out = _lut(table, index)
    scale = jnp.array([2.0 / 256.0], dtype=jnp.float32)
    return (out, scale)
